# baseline (speedup 1.0000x reference)
import jax
import jax.numpy as jnp
from jax.experimental import pallas as pl
from jax.experimental.pallas import tpu as pltpu

N_EXP = 8
D_MODEL = 768
BT = 2048


def _route(tok, wn, w_out, idx_out):
    tnorm = jnp.sqrt(jnp.sum(tok * tok, axis=-1, keepdims=True))
    nt = (tok / jnp.maximum(tnorm, 1e-12)).astype(jnp.bfloat16)
    sims = jax.lax.dot_general(
        nt, wn, dimension_numbers=(((1,), (1,)), ((), ())),
        preferred_element_type=jnp.float32)
    m = jnp.max(sims, axis=-1, keepdims=True)
    eiota = jax.lax.broadcasted_iota(jnp.int32, sims.shape, 1)
    idx = jnp.min(jnp.where(sims == m, eiota, N_EXP), axis=-1, keepdims=True)
    w_out[...] = (eiota == idx).astype(jnp.float32)
    idx_out[...] = idx


def _gate_body(tok_a_ref, tok_b_ref, emb_ref, w_ref, idx_ref):
    emb = emb_ref[...]
    norm = jnp.sqrt(jnp.sum(emb * emb, axis=-1, keepdims=True))
    wn = (emb / jnp.maximum(norm, 1e-12)).astype(jnp.bfloat16)
    _route(tok_a_ref[0], wn, w_ref.at[0], idx_ref.at[0])
    _route(tok_b_ref[0], wn, w_ref.at[1], idx_ref.at[1])


@jax.jit
def kernel(language_token, routing_embeddings):
    n_tokens = language_token.shape[0]
    half = n_tokens // 2
    steps = half // BT
    tok3 = language_token.reshape(2, half, D_MODEL)
    weights, indices = pl.pallas_call(
        _gate_body,
        grid=(steps,),
        in_specs=[
            pl.BlockSpec((1, BT, D_MODEL), lambda i: (0, i, 0)),
            pl.BlockSpec((1, BT, D_MODEL), lambda i: (1, i, 0)),
            pl.BlockSpec((N_EXP, D_MODEL), lambda i: (0, 0)),
        ],
        out_specs=[
            pl.BlockSpec((2, BT, N_EXP), lambda i: (0, i, 0)),
            pl.BlockSpec((2, BT, 1), lambda i: (0, i, 0)),
        ],
        out_shape=[
            jax.ShapeDtypeStruct((2, half, N_EXP), jnp.float32),
            jax.ShapeDtypeStruct((2, half, 1), jnp.int32),
        ],
    )(tok3, tok3, routing_embeddings)
    return (weights.reshape(n_tokens, N_EXP), indices.reshape(n_tokens, 1))
